# Initial kernel scaffold; baseline (speedup 1.0000x reference)
import jax
import jax.numpy as jnp
from jax.experimental import pallas as pl


def kernel(x, edge_index, W_in, b_in, Wl, bl, Wr, gamma, beta, W_out, b_out):
    raise NotImplementedError("write your pallas kernel here")


# speedup vs baseline: 1.0109x; 1.0109x over previous
import functools

import jax
import jax.numpy as jnp
from jax import lax
from jax.experimental import pallas as pl

_EPS = 1e-5
_BN = 1000


def _in_proj_kernel(x_ref, w_ref, b_ref, o_ref):
    acc = lax.dot_general(x_ref[...], w_ref[...],
                          (((1,), (1,)), ((), ())),
                          preferred_element_type=jnp.float32)
    o_ref[...] = jnp.maximum(acc + b_ref[...], 0.0)


def _layer_kernel(agg_ref, cnt_ref, h_ref, wl_ref, bl_ref, wr_ref,
                  g_ref, be_ref, o_ref):
    cnt = jnp.maximum(cnt_ref[...], 1.0)
    agg = agg_ref[...] / cnt
    h = h_ref[...]
    t = lax.dot_general(agg, wl_ref[...], (((1,), (1,)), ((), ())),
                        preferred_element_type=jnp.float32)
    t = t + bl_ref[...]
    t = t + lax.dot_general(h, wr_ref[...], (((1,), (1,)), ((), ())),
                            preferred_element_type=jnp.float32)
    mu = jnp.mean(t, axis=-1, keepdims=True)
    d = t - mu
    var = jnp.mean(d * d, axis=-1, keepdims=True)
    t = d * lax.rsqrt(var + _EPS) * g_ref[...] + be_ref[...]
    o_ref[...] = jnp.maximum(t, 0.0) + h


def _out_proj_kernel(h_ref, w_ref, b_ref, o_ref):
    o_ref[...] = jnp.sum(h_ref[...] * w_ref[...], axis=-1,
                         keepdims=True) + b_ref[...]


def _row_blocked(kern, n, h, extra_specs, out_cols):
    grid = (n // _BN,)
    return pl.pallas_call(
        kern,
        grid=grid,
        in_specs=[pl.BlockSpec((_BN, h), lambda i: (i, 0))] + extra_specs,
        out_specs=pl.BlockSpec((_BN, out_cols), lambda i: (i, 0)),
        out_shape=jax.ShapeDtypeStruct((n, out_cols), jnp.float32),
    )


def kernel(x, edge_index, W_in, b_in, Wl, bl, Wr, gamma, beta, W_out, b_out):
    n, in_dim = x.shape
    h_dim = W_in.shape[0]
    L = Wl.shape[0]
    src = edge_index[0]
    dst = edge_index[1]

    wspec = pl.BlockSpec((h_dim, in_dim), lambda i: (0, 0))
    vspec = pl.BlockSpec((1, h_dim), lambda i: (0, 0))

    h = _row_blocked(_in_proj_kernel, n, in_dim,
                     [wspec, vspec], h_dim)(x, W_in, b_in.reshape(1, h_dim))

    cnt = jax.ops.segment_sum(jnp.ones((e := src.shape[0], 1), jnp.float32),
                              dst, num_segments=n)

    layer = pl.pallas_call(
        _layer_kernel,
        grid=(n // _BN,),
        in_specs=[
            pl.BlockSpec((_BN, h_dim), lambda i: (i, 0)),
            pl.BlockSpec((_BN, 1), lambda i: (i, 0)),
            pl.BlockSpec((_BN, h_dim), lambda i: (i, 0)),
            pl.BlockSpec((h_dim, h_dim), lambda i: (0, 0)),
            vspec,
            pl.BlockSpec((h_dim, h_dim), lambda i: (0, 0)),
            vspec, vspec,
        ],
        out_specs=pl.BlockSpec((_BN, h_dim), lambda i: (i, 0)),
        out_shape=jax.ShapeDtypeStruct((n, h_dim), jnp.float32),
    )

    for i in range(L):
        agg = jax.ops.segment_sum(jnp.take(h, src, axis=0), dst,
                                  num_segments=n)
        h = layer(agg, cnt, h, Wl[i], bl[i].reshape(1, h_dim), Wr[i],
                  gamma[i].reshape(1, h_dim), beta[i].reshape(1, h_dim))

    out = _row_blocked(_out_proj_kernel, n, h_dim,
                       [vspec, pl.BlockSpec((1, 1), lambda i: (0, 0))], 1)(
        h, W_out.reshape(1, h_dim), b_out.reshape(1, 1))
    return out.reshape(-1)
